# Initial kernel scaffold; baseline (speedup 1.0000x reference)
import jax
import jax.numpy as jnp
from jax.experimental import pallas as pl


def kernel(x_operation, x_machine, x_job, edge_attr_precedence, edge_attr_assigned_to, edge_attr_can_process, edge_attr_contains, edge_attr_belongs_to, params, edge_index_precedence, edge_index_assigned_to, edge_index_can_process, edge_index_contains, edge_index_belongs_to):
    raise NotImplementedError("write your pallas kernel here")


# speedup vs baseline: 8.6037x; 8.6037x over previous
import functools
import jax
import jax.numpy as jnp
from jax.experimental import pallas as pl

HID = 128
HEADS = 8
ETS = [("operation", "precedence", "operation"), ("operation", "assigned_to", "machine"), ("machine", "can_process", "operation"), ("job", "contains", "operation"), ("operation", "belongs_to", "job")]
NODE_TYPES = ["operation", "machine", "job"]
RELS = [et[1] for et in ETS]


def _mm_kernel(x_ref, w_ref, o_ref):
    o_ref[...] = jax.lax.dot_general(
        x_ref[...], w_ref[...], (((1,), (0,)), ((), ())),
        preferred_element_type=jnp.float32)


def _mm(x, w):
    N, K = x.shape
    M = w.shape[1]
    Np = (N + 255) // 256 * 256
    Kp = (K + 127) // 128 * 128
    Mp = (M + 127) // 128 * 128
    xp = jnp.pad(x, ((0, Np - N), (0, Kp - K)))
    wp = jnp.pad(w, ((0, Kp - K), (0, Mp - M)))
    out = pl.pallas_call(
        _mm_kernel,
        grid=(Np // 256, Mp // 128),
        in_specs=[pl.BlockSpec((256, Kp), lambda i, j: (i, 0)),
                  pl.BlockSpec((Kp, 128), lambda i, j: (0, j))],
        out_specs=pl.BlockSpec((256, 128), lambda i, j: (i, j)),
        out_shape=jax.ShapeDtypeStruct((Np, Mp), jnp.float32),
    )(xp, wp)
    return out[:N, :M]


def _edge_phase(hs, asv, adv, aev, src, dst, n_dst, heads):
    c = HID // heads
    M = jax.nn.leaky_relu(asv.max(0) + adv.max(0) + aev.max(0), 0.2)
    logit = jax.nn.leaky_relu(asv[src] + adv[dst] + aev, 0.2)
    ex = jnp.exp(logit - M)
    w = jnp.repeat(ex, c, axis=1)
    num = jax.ops.segment_sum(hs[src] * w, dst, num_segments=n_dst)
    den = jax.ops.segment_sum(ex, dst, num_segments=n_dst)
    return num, den


def kernel(x_operation, x_machine, x_job, edge_attr_precedence, edge_attr_assigned_to, edge_attr_can_process, edge_attr_contains, edge_attr_belongs_to, params, edge_index_precedence, edge_index_assigned_to, edge_index_can_process, edge_index_contains, edge_index_belongs_to):
    x_in = {"operation": x_operation, "machine": x_machine, "job": x_job}
    eattrs = {"precedence": edge_attr_precedence, "assigned_to": edge_attr_assigned_to, "can_process": edge_attr_can_process, "contains": edge_attr_contains, "belongs_to": edge_attr_belongs_to}
    eidx = {"precedence": edge_index_precedence, "assigned_to": edge_index_assigned_to, "can_process": edge_index_can_process, "contains": edge_index_contains, "belongs_to": edge_index_belongs_to}
    n_nodes = {t: x_in[t].shape[0] for t in NODE_TYPES}

    x = {t: _mm(x_in[t], params["proj"][t]["W"]) + params["proj"][t]["b"]
         for t in NODE_TYPES}

    aev_all = {}
    for rel in RELS:
        Wep, bep = params["eproj"][rel]["W"], params["eproj"][rel]["b"]
        folds, bias_folds, splits = [], [], []
        for l in range(3):
            heads = HEADS if l < 2 else 1
            c = HID // heads
            p = params["convs"][l][rel]
            A_edge = jnp.einsum("khc,hc->kh", p["W_edge"].reshape(HID, heads, c), p["att_edge"])
            folds.append(Wep @ A_edge)
            bias_folds.append(bep @ A_edge)
            splits.append(heads)
        Wf = jnp.concatenate(folds, axis=1)
        bf = jnp.concatenate(bias_folds, axis=0)
        full = _mm(eattrs[rel], Wf) + bf
        o0 = 0
        per_layer = []
        for h in splits:
            per_layer.append(full[:, o0:o0 + h])
            o0 += h
        aev_all[rel] = per_layer

    inv = 1.0 / jnp.sqrt(1.0 + 1e-5)
    for l in range(3):
        heads = HEADS if l < 2 else 1
        c = HID // heads
        res = x
        cols = {t: [] for t in NODE_TYPES}
        layouts = {t: [] for t in NODE_TYPES}
        for (s, rel, d) in ETS:
            p = params["convs"][l][rel]
            A_src = jnp.einsum("khc,hc->kh", p["W"].reshape(HID, heads, c), p["att_src"])
            A_dst = jnp.einsum("khc,hc->kh", p["W"].reshape(HID, heads, c), p["att_dst"])
            cols[s].append(p["W"]); layouts[s].append(("hs", rel, HID))
            cols[s].append(A_src); layouts[s].append(("as", rel, heads))
            cols[d].append(A_dst); layouts[d].append(("ad", rel, heads))
        hs_all, asv_all, adv_all = {}, {}, {}
        for t in NODE_TYPES:
            big = _mm(x[t], jnp.concatenate(cols[t], axis=1))
            o0 = 0
            for kind, rel, width in layouts[t]:
                blk = big[:, o0:o0 + width]
                o0 += width
                if kind == "hs":
                    hs_all[rel] = blk
                elif kind == "as":
                    asv_all[rel] = blk
                else:
                    adv_all[rel] = blk
        agg = {t: jnp.zeros((n_nodes[t], HID), jnp.float32) for t in NODE_TYPES}
        bias_sum = {t: jnp.zeros((HID,), jnp.float32) for t in NODE_TYPES}
        for (s, rel, d) in ETS:
            p = params["convs"][l][rel]
            src, dst = eidx[rel][0], eidx[rel][1]
            num, den = _edge_phase(hs_all[rel], asv_all[rel], adv_all[rel],
                                   aev_all[rel][l], src, dst, n_nodes[d], heads)
            o = num / jnp.repeat(den + 1e-16, c, axis=1)
            agg[d] = agg[d] + o
            bias_sum[d] = bias_sum[d] + p["bias"]
        nx = {}
        for t in NODE_TYPES:
            g = params["bn"][l][t]["gamma"]
            b = params["bn"][l][t]["beta"]
            h = (agg[t] + bias_sum[t]) * (inv * g) + b
            if l < 2:
                h = jax.nn.relu(h)
            nx[t] = h + res[t]
        x = nx
    return _mm(x["operation"], params["out"]["W"]) + params["out"]["b"]


# speedup vs baseline: 16.3712x; 1.9028x over previous
import functools
import jax
import jax.numpy as jnp
from jax import lax
from jax.experimental import pallas as pl
from jax.experimental.pallas import tpu as pltpu
from jax.experimental.pallas import tpu_sc as plsc

HID = 128
HEADS = 8
ETS = [("operation", "precedence", "operation"), ("operation", "assigned_to", "machine"), ("machine", "can_process", "operation"), ("job", "contains", "operation"), ("operation", "belongs_to", "job")]
NODE_TYPES = ["operation", "machine", "job"]
RELS = [et[1] for et in ETS]

NC, NS, LANES = 2, 16, 16
E_PAD = 102400


def _mm_kernel(x_ref, w_ref, o_ref):
    o_ref[...] = lax.dot_general(
        x_ref[...], w_ref[...], (((1,), (0,)), ((), ())),
        preferred_element_type=jnp.float32)


def _mm(x, w):
    N, K = x.shape
    M = w.shape[1]
    Np = (N + 255) // 256 * 256
    Kp = (K + 127) // 128 * 128
    Mp = (M + 127) // 128 * 128
    xp = jnp.pad(x, ((0, Np - N), (0, Kp - K)))
    wp = jnp.pad(w, ((0, Kp - K), (0, Mp - M)))
    out = pl.pallas_call(
        _mm_kernel,
        grid=(Np // 256, Mp // 128),
        in_specs=[pl.BlockSpec((256, Kp), lambda i, j: (i, 0)),
                  pl.BlockSpec((Kp, 128), lambda i, j: (0, j))],
        out_specs=pl.BlockSpec((256, 128), lambda i, j: (i, j)),
        out_shape=jax.ShapeDtypeStruct((Np, Mp), jnp.float32),
    )(xp, wp)
    return out[:N, :M]


def _phase_a(src_flat, dst_flat, aev, asv, adv, m_vec):
    n_groups = E_PAD // 128
    per_tile = n_groups // (NC * NS)
    ept = per_tile * 128
    mesh = plsc.VectorSubcoreMesh(core_axis_name="c", subcore_axis_name="s")
    buf_t = [pltpu.VMEM((128, 16), jnp.float32) for _ in range(8)]

    @functools.partial(
        pl.kernel, mesh=mesh,
        compiler_params=pltpu.CompilerParams(use_tc_tiling_on_sc=False, needs_layout_passes=False),
        out_type=jax.ShapeDtypeStruct((E_PAD, 16), jnp.float32),
        scratch_types=[
            pltpu.VMEM((ept,), jnp.int32),
            pltpu.VMEM((ept,), jnp.int32),
            *buf_t,
            pltpu.VMEM((16,), jnp.float32),
            pltpu.SemaphoreType.DMA,
            pltpu.SemaphoreType.DMA,
            pltpu.SemaphoreType.DMA,
            pltpu.SemaphoreType.DMA,
        ])
    def k(src_h, dst_h, aev_h, asv_h, adv_h, m_h, ex_h,
          srcb, dstb, asg0, adg0, aeg0, exb0, asg1, adg1, aeg1, exb1,
          mv, semg0, semg1, semo0, semo1):
        wid = lax.axis_index("s") * NC + lax.axis_index("c")
        tb = wid * ept
        pltpu.sync_copy(m_h, mv)
        pltpu.sync_copy(src_h.at[pl.ds(tb, ept)], srcb)
        pltpu.sync_copy(dst_h.at[pl.ds(tb, ept)], dstb)
        m = mv[...]
        bufs = [(asg0, adg0, aeg0, exb0, semg0, semo0),
                (asg1, adg1, aeg1, exb1, semg1, semo1)]

        def fire(g, B):
            asg, adg, aeg, _, semg, _ = B
            cps = [
                pltpu.async_copy(asv_h.at[srcb.at[pl.ds(g * 128, 128)]], asg, semg),
                pltpu.async_copy(adv_h.at[dstb.at[pl.ds(g * 128, 128)]], adg, semg),
                pltpu.async_copy(aev_h.at[pl.ds(tb + g * 128, 128)], aeg, semg),
            ]
            return cps

        pend_g = {0: fire(0, bufs[0])}
        pend_o = {}
        for g in range(per_tile):
            B = bufs[g % 2]
            asg, adg, aeg, exb, _, semo = B
            if g + 1 < per_tile:
                pend_g[g + 1] = fire(g + 1, bufs[(g + 1) % 2])
            for cp in pend_g.pop(g):
                cp.wait()
            if g - 2 in pend_o:
                pend_o.pop(g - 2).wait()

            def inner(i, _):
                for u in range(4):
                    e = i * 4 + u
                    z = asg[e] + adg[e] + aeg[e]
                    z = jnp.maximum(z, z * 0.2) - m
                    exb[e] = jnp.exp(z)
                return 0

            lax.fori_loop(0, 32, inner, 0)
            pend_o[g] = pltpu.async_copy(exb, ex_h.at[pl.ds(tb + g * 128, 128)], semo)
        for cp in pend_o.values():
            cp.wait()

    return k(src_flat, dst_flat, aev, asv, adv, m_vec)


def _phase_b(src_flat, dst_flat, ex, hs, heads, n_dst, chunks_per_core, R,
             split_edges_by_core, compact=True):
    c_sz = HID // heads
    widx = [min(h8 * LANES // c_sz, heads - 1) for h8 in range(HID // LANES)]
    ept = E_PAD // NS // (NC if split_edges_by_core else 1)
    rows_pt = R // NS
    assert rows_pt % 16 == 0 and ept % 128 == 0
    if split_edges_by_core:
        out_t = (jax.ShapeDtypeStruct((NC, R, HID), jnp.float32),
                 jax.ShapeDtypeStruct((NC, R, 16), jnp.float32))
    else:
        out_t = (jax.ShapeDtypeStruct((NC * chunks_per_core * R, HID), jnp.float32),
                 jax.ShapeDtypeStruct((NC * chunks_per_core * R, 16), jnp.float32))
    _dn = lax.GatherDimensionNumbers(offset_dims=(), collapsed_slice_dims=(0,),
                                     start_index_map=(0,))

    def _splat(v, lane):
        idxcol = lax.broadcast(jnp.int32(lane), (16, 1))
        return lax.gather(v, idxcol, _dn, (1,),
                          mode=lax.GatherScatterMode.PROMISE_IN_BOUNDS)
    mesh = plsc.VectorSubcoreMesh(core_axis_name="c", subcore_axis_name="s")

    @functools.partial(
        pl.kernel, mesh=mesh, out_type=out_t,
        compiler_params=pltpu.CompilerParams(use_tc_tiling_on_sc=False, needs_layout_passes=False),
        scratch_types=[
            pltpu.VMEM((ept,), jnp.int32),
            pltpu.VMEM((ept,), jnp.int32),
            pltpu.VMEM((ept + 16,), jnp.int32),
            pltpu.VMEM((ept + 16,), jnp.int32),
            pltpu.VMEM((128,), jnp.int32),
            pltpu.VMEM((128,), jnp.int32),
            pltpu.VMEM((128,), jnp.int32),
            pltpu.VMEM((128, 16), jnp.float32),
            pltpu.VMEM((128, HID), jnp.float32),
            pltpu.VMEM((16, HID), jnp.float32),
            pltpu.VMEM((16, 16), jnp.float32),
            pltpu.VMEM_SHARED((R + 8, HID), jnp.float32),
            pltpu.VMEM_SHARED((R + 8, 16), jnp.float32),
            pltpu.SemaphoreType.DMA,
            pltpu.SemaphoreType.DMA,
        ])
    def k(src_h, dst_h, ex_h, hs_h, num_h, den_h,
          srcb, dstb, cidx, ldx, sidx, gidx, lix, exg, rows, zb, zbd,
          num_sp, den_sp, sem1, sem2):
        c = lax.axis_index("c")
        s = lax.axis_index("s")
        ebase = (c * (E_PAD // NC) if split_edges_by_core else 0) + s * ept
        pltpu.sync_copy(src_h.at[pl.ds(ebase, ept)], srcb)
        pltpu.sync_copy(dst_h.at[pl.ds(ebase, ept)], dstb)
        zero16 = jnp.zeros((16,), jnp.float32)
        for i in range(16):
            zbd[i] = zero16
            for kk in range(HID // 16):
                zb[i, pl.ds(kk * 16, 16)] = zero16

        for ci in range(chunks_per_core):
            if split_edges_by_core:
                lo = 0
            else:
                lo = (c * chunks_per_core + ci) * R

            zcps = []
            for zi in range(rows_pt // 16):
                zcps.append(pltpu.async_copy(
                    zb, num_sp.at[pl.ds(s * rows_pt + zi * 16, 16)], sem1))
                zcps.append(pltpu.async_copy(
                    zbd, den_sp.at[pl.ds(s * rows_pt + zi * 16, 16)], sem2))
            for zcp in zcps:
                zcp.wait()
            plsc.subcore_barrier()

            r_fill = jnp.full((16,), R, jnp.int32)
            zero16i = jnp.zeros((16,), jnp.int32)

            def pfbody(i, _):
                for u in range(4):
                    ldx[pl.ds((i * 4 + u) * 16, 16)] = r_fill
                    cidx[pl.ds((i * 4 + u) * 16, 16)] = zero16i
                return 0
            lax.fori_loop(0, ept // 64, pfbody, 0)
            ldx[pl.ds(ept, 16)] = r_fill
            cidx[pl.ds(ept, 16)] = zero16i

            iota16 = lax.iota(jnp.int32, 16)

            if not compact:
                def gblk(g, _):
                    for kk in range(8):
                        dv = dstb[pl.ds(g * 128 + kk * 16, 16)] - lo
                        msk = (dv >= 0) & (dv < R)
                        lix[pl.ds(kk * 16, 16)] = jnp.where(msk, dv, R)
                        sidx[pl.ds(kk * 16, 16)] = srcb[pl.ds(g * 128 + kk * 16, 16)]
                    cp1 = pltpu.async_copy(hs_h.at[sidx], rows, sem1)
                    pltpu.sync_copy(ex_h.at[pl.ds(ebase + g * 128, 128)], exg)
                    cp1.wait()

                    def scale(i, _):
                        for u in range(4):
                            e = i * 4 + u
                            er = exg[e]
                            for h8 in range(HID // 16):
                                w = _splat(er, widx[h8])
                                rows[e, pl.ds(h8 * 16, 16)] = (
                                    rows[e, pl.ds(h8 * 16, 16)] * w)
                        return 0
                    lax.fori_loop(0, 32, scale, 0)
                    pltpu.sync_copy(rows, num_sp.at[lix], add=True)
                    pltpu.sync_copy(exg, den_sp.at[lix], add=True)
                    return 0
                lax.fori_loop(0, ept // 128, gblk, 0)
                plsc.subcore_barrier()
                if split_edges_by_core:
                    pltpu.sync_copy(num_sp.at[pl.ds(s * rows_pt, rows_pt)],
                                    num_h.at[c, pl.ds(s * rows_pt, rows_pt)])
                    pltpu.sync_copy(den_sp.at[pl.ds(s * rows_pt, rows_pt)],
                                    den_h.at[c, pl.ds(s * rows_pt, rows_pt)])
                else:
                    pltpu.sync_copy(num_sp.at[pl.ds(s * rows_pt, rows_pt)],
                                    num_h.at[pl.ds(lo + s * rows_pt, rows_pt)])
                    pltpu.sync_copy(den_sp.at[pl.ds(s * rows_pt, rows_pt)],
                                    den_h.at[pl.ds(lo + s * rows_pt, rows_pt)])
                plsc.subcore_barrier()
                continue

            def scan_body(i, basev):
                for u in range(4):
                    d = dstb[pl.ds((i * 4 + u) * 16, 16)] - lo
                    msk = (d >= 0) & (d < R)
                    ones = jnp.where(msk, 1, 0).astype(jnp.int32)
                    pos = basev + plsc.cumsum(ones) - 1
                    pos = jnp.where(msk, pos, ept)
                    eid = iota16 + (i * 4 + u) * 16
                    plsc.store_scatter(cidx, [pos], eid)
                    plsc.store_scatter(ldx, [pos], jnp.where(msk, d, R))
                    basev = basev + plsc.all_reduce_population_count(msk)
                return basev

            basev = lax.fori_loop(0, ept // 64, scan_body,
                                  jnp.zeros((16,), jnp.int32))
            n = jnp.max(basev)
            nb = (n + 127) // 128

            def blk(b, _):
                boff = b * 128
                for kk in range(8):
                    cv = cidx[pl.ds(boff + kk * 16, 16)]
                    sidx[pl.ds(kk * 16, 16)] = plsc.load_gather(srcb, [cv])
                    gidx[pl.ds(kk * 16, 16)] = cv + ebase
                    lix[pl.ds(kk * 16, 16)] = ldx[pl.ds(boff + kk * 16, 16)]
                cp1 = pltpu.async_copy(hs_h.at[sidx], rows, sem1)
                cp2 = pltpu.async_copy(ex_h.at[gidx], exg, sem2)
                cp1.wait()
                cp2.wait()

                def scale(i, _):
                    for u in range(4):
                        e = i * 4 + u
                        er = exg[e]
                        for h8 in range(HID // 16):
                            w = _splat(er, widx[h8])
                            rows[e, pl.ds(h8 * 16, 16)] = (
                                rows[e, pl.ds(h8 * 16, 16)] * w)
                    return 0
                lax.fori_loop(0, 32, scale, 0)
                pltpu.sync_copy(rows, num_sp.at[lix], add=True)
                pltpu.sync_copy(exg, den_sp.at[lix], add=True)
                return 0

            lax.fori_loop(0, nb, blk, 0)
            plsc.subcore_barrier()
            if split_edges_by_core:
                pltpu.sync_copy(num_sp.at[pl.ds(s * rows_pt, rows_pt)],
                                num_h.at[c, pl.ds(s * rows_pt, rows_pt)])
                pltpu.sync_copy(den_sp.at[pl.ds(s * rows_pt, rows_pt)],
                                den_h.at[c, pl.ds(s * rows_pt, rows_pt)])
            else:
                pltpu.sync_copy(num_sp.at[pl.ds(s * rows_pt, rows_pt)],
                                num_h.at[pl.ds(lo + s * rows_pt, rows_pt)])
                pltpu.sync_copy(den_sp.at[pl.ds(s * rows_pt, rows_pt)],
                                den_h.at[pl.ds(lo + s * rows_pt, rows_pt)])
            plsc.subcore_barrier()

    return k(src_flat, dst_flat, ex, hs)


def _combine(terms, rep, gscale, bshift, res, relu):
    Np = res.shape[0]
    n_args = []
    struct = []
    for nums, dens in terms:
        struct.append((len(nums), len(dens)))
        n_args.extend(nums)
        n_args.extend(dens)

    all_in = n_args + [res, rep, gscale, bshift]
    in_specs = []
    for a in n_args:
        w = a.shape[1]
        in_specs.append(pl.BlockSpec((256, w), lambda i: (i, 0)))
    in_specs.append(pl.BlockSpec((256, HID), lambda i: (i, 0)))
    in_specs.append(pl.BlockSpec((16, HID), lambda i: (0, 0)))
    in_specs.append(pl.BlockSpec((1, HID), lambda i: (0, 0)))
    in_specs.append(pl.BlockSpec((1, HID), lambda i: (0, 0)))

    def kern(*refs):
        args = refs[:-1]
        o_ref = refs[-1]
        pos = 0
        acc = None
        for n_n, n_d in struct:
            num = args[pos][...]
            for j in range(1, n_n):
                num = num + args[pos + j][...]
            den = args[pos + n_n][...]
            for j in range(1, n_d):
                den = den + args[pos + n_n + j][...]
            pos += n_n + n_d
            den_rep = lax.dot_general(den, args[pos_rep][...],
                                      (((1,), (0,)), ((), ())),
                                      preferred_element_type=jnp.float32)
            term = num / (den_rep + 1e-16)
            acc = term if acc is None else acc + term
        h = acc * args[pos_g][...] + args[pos_b][...]
        if relu:
            h = jnp.maximum(h, 0.0)
        o_ref[...] = h + args[pos_res][...]

    pos_res = len(n_args)
    pos_rep = len(n_args) + 1
    pos_g = len(n_args) + 2
    pos_b = len(n_args) + 3
    return pl.pallas_call(
        kern,
        grid=(Np // 256,),
        in_specs=in_specs,
        out_specs=pl.BlockSpec((256, HID), lambda i: (i, 0)),
        out_shape=jax.ShapeDtypeStruct((Np, HID), jnp.float32),
    )(*all_in)


def _pad_rows(x, rows):
    return jnp.pad(x, ((0, rows - x.shape[0]), (0, 0)))


def _att_fold(W, att, heads, c):
    return jnp.einsum("khc,hc->kh", W.reshape(HID, heads, c), att)


def kernel(x_operation, x_machine, x_job, edge_attr_precedence, edge_attr_assigned_to, edge_attr_can_process, edge_attr_contains, edge_attr_belongs_to, params, edge_index_precedence, edge_index_assigned_to, edge_index_can_process, edge_index_contains, edge_index_belongs_to):
    x_in = {"operation": x_operation, "machine": x_machine, "job": x_job}
    eattrs = {"precedence": edge_attr_precedence, "assigned_to": edge_attr_assigned_to, "can_process": edge_attr_can_process, "contains": edge_attr_contains, "belongs_to": edge_attr_belongs_to}
    eidx = {"precedence": edge_index_precedence, "assigned_to": edge_index_assigned_to, "can_process": edge_index_can_process, "contains": edge_index_contains, "belongs_to": edge_index_belongs_to}
    n_nodes = {t: x_in[t].shape[0] for t in NODE_TYPES}
    E = eidx["precedence"].shape[1]

    src_flat, dst_flat = {}, {}
    for (s, rel, d) in ETS:
        src_flat[rel] = jnp.concatenate(
            [eidx[rel][0], jnp.zeros((E_PAD - E,), jnp.int32)])
        dst_flat[rel] = jnp.concatenate(
            [eidx[rel][1], jnp.full((E_PAD - E,), n_nodes[d], jnp.int32)])

    x = {t: _mm(x_in[t], params["proj"][t]["W"]) + params["proj"][t]["b"]
         for t in NODE_TYPES}

    aev_all = {rel: [] for rel in RELS}
    for rel in RELS:
        Wep, bep = params["eproj"][rel]["W"], params["eproj"][rel]["b"]
        folds, bias_folds, widths = [], [], []
        for l in range(3):
            heads = HEADS if l < 2 else 1
            c = HID // heads
            p = params["convs"][l][rel]
            A_edge = _att_fold(p["W_edge"], p["att_edge"], heads, c)
            folds.append(Wep @ A_edge)
            bias_folds.append(bep @ A_edge)
            widths.append(heads)
        full = _mm(eattrs[rel], jnp.concatenate(folds, axis=1)) + \
            jnp.concatenate(bias_folds, axis=0)
        o0 = 0
        for heads in widths:
            blk = full[:, o0:o0 + heads]
            o0 += heads
            blk = jnp.pad(blk, ((0, E_PAD - E), (0, 16 - heads)))
            aev_all[rel].append(blk)

    R_OP, CPC_OP = 8448, 3
    R_SM = 5120
    inv = 1.0 / jnp.sqrt(1.0 + 1e-5)

    for l in range(3):
        heads = HEADS if l < 2 else 1
        c = HID // heads
        last = l == 2
        ets_l = [e for e in ETS if (not last or e[2] == "operation")]
        rels_l = [e[1] for e in ets_l]
        res = x
        cols = {t: [] for t in NODE_TYPES}
        layouts = {t: [] for t in NODE_TYPES}
        for (s, rel, d) in ets_l:
            p = params["convs"][l][rel]
            cols[s].append(p["W"]); layouts[s].append(("hs", rel, HID))
            cols[s].append(_att_fold(p["W"], p["att_src"], heads, c))
            layouts[s].append(("as", rel, heads))
            cols[d].append(_att_fold(p["W"], p["att_dst"], heads, c))
            layouts[d].append(("ad", rel, heads))
        hs_all, asv_all, adv_all = {}, {}, {}
        for t in NODE_TYPES:
            if not cols[t]:
                continue
            big = _mm(x[t], jnp.concatenate(cols[t], axis=1))
            o0 = 0
            for kind, rel, width in layouts[t]:
                blk = big[:, o0:o0 + width]
                o0 += width
                if kind == "hs":
                    hs_all[rel] = blk
                elif kind == "as":
                    asv_all[rel] = blk
                else:
                    adv_all[rel] = blk

        numden = {}
        for (s, rel, d) in ets_l:
            asv, adv = asv_all[rel], adv_all[rel]
            aev = aev_all[rel][l]
            m8 = jnp.max(asv, 0) + jnp.max(adv, 0) + jnp.max(aev[:E, :heads], 0)
            m8 = jnp.maximum(m8, m8 * 0.2)
            m16 = jnp.pad(m8, (0, 16 - heads))
            asv_p = _pad_rows(jnp.pad(asv, ((0, 0), (0, 16 - heads))),
                              (n_nodes[s] + 15) // 8 * 8)
            adv_p = _pad_rows(jnp.pad(adv, ((0, 0), (0, 16 - heads))),
                              (n_nodes[d] + 15) // 8 * 8)
            ex = _phase_a(src_flat[rel], dst_flat[rel], aev, asv_p, adv_p, m16)
            if d == "operation":
                num, den = _phase_b(src_flat[rel], dst_flat[rel], ex,
                                    hs_all[rel], heads, n_nodes[d],
                                    CPC_OP, R_OP, False)
                numden[rel] = ([num], [den])
            else:
                num, den = _phase_b(src_flat[rel], dst_flat[rel], ex,
                                    hs_all[rel], heads, n_nodes[d],
                                    1, R_SM, True)
                numden[rel] = ([num[0], num[1]], [den[0], den[1]])

        rep = jnp.zeros((16, HID), jnp.float32)
        for h in range(heads):
            rep = rep.at[h, h * c:(h + 1) * c].set(1.0)
        nx = {}
        types_l = ["operation"] if last else NODE_TYPES
        for t in types_l:
            rls = [rel for (s_, rel, d_) in ets_l if d_ == t]
            terms = [numden[rel] for rel in rls]
            np_rows = terms[0][0][0].shape[0]
            bias_sum = sum(params["convs"][l][rel]["bias"] for rel in rls)
            g = params["bn"][l][t]["gamma"]
            b = params["bn"][l][t]["beta"]
            gscale = (inv * g).reshape(1, HID)
            bshift = (bias_sum * inv * g + b).reshape(1, HID)
            res_p = _pad_rows(res[t], np_rows)
            out = _combine(terms, rep, gscale, bshift, res_p, relu=not last)
            nx[t] = out[:n_nodes[t]]
        for t in NODE_TYPES:
            if t not in nx:
                nx[t] = res[t]
        x = nx
    return _mm(x["operation"], params["out"]["W"]) + params["out"]["b"]
